# baseline (speedup 1.0000x reference)
import functools

import jax
import jax.numpy as jnp
from jax import lax
from jax.experimental import pallas as pl
from jax.experimental.pallas import tpu as pltpu

_BLK = 4096


def _assign_body(x_ref, w_ref, out_ref):
    xb = x_ref[...]
    wb = w_ref[...]
    dots = lax.dot_general(xb, wb, (((1,), (1,)), ((), ())),
                           preferred_element_type=jnp.float32)
    x2 = jnp.sum(xb * xb, axis=1, keepdims=True)
    w2 = jnp.sum(wb * wb, axis=1)[None, :]
    d2 = jnp.maximum(x2 + w2 - 2.0 * dots, 0.0)
    k = d2.shape[1]
    min_d = jnp.min(d2, axis=1, keepdims=True)
    iota_k = lax.broadcasted_iota(jnp.int32, d2.shape, 1)
    idx = jnp.min(jnp.where(d2 == min_d, iota_k, k), axis=1)
    out_ref[...] = idx.astype(jnp.int32)


@jax.jit
def kernel(x, weights):
    b, d = x.shape
    grid = (b // _BLK,)
    return pl.pallas_call(
        _assign_body,
        grid=grid,
        in_specs=[
            pl.BlockSpec((_BLK, d), lambda i: (i, 0)),
            pl.BlockSpec(weights.shape, lambda i: (0, 0)),
        ],
        out_specs=pl.BlockSpec((_BLK,), lambda i: (i,)),
        out_shape=jax.ShapeDtypeStruct((b,), jnp.int32),
        compiler_params=pltpu.CompilerParams(
            dimension_semantics=("arbitrary",),
        ),
    )(x, weights)


# speedup vs baseline: 2.6243x; 2.6243x over previous
import functools

import jax
import jax.numpy as jnp
from jax import lax
from jax.experimental import pallas as pl
from jax.experimental.pallas import tpu as pltpu

_BLK = 4096


def _assign_body(x_ref, w_ref, out_ref):
    xb = x_ref[...]
    wb = w_ref[...]
    dots = lax.dot_general(wb, xb, (((1,), (1,)), ((), ())),
                           preferred_element_type=jnp.float32)
    w2 = jnp.sum(wb * wb, axis=1, keepdims=True)
    s = w2 - 2.0 * dots
    k = s.shape[0]
    min_s = jnp.min(s, axis=0, keepdims=True)
    iota_k = lax.broadcasted_iota(jnp.int32, s.shape, 0)
    idx = jnp.min(jnp.where(s == min_s, iota_k, k), axis=0)
    out_ref[...] = idx.astype(jnp.int32)


@jax.jit
def kernel(x, weights):
    b, d = x.shape
    grid = (b // _BLK,)
    return pl.pallas_call(
        _assign_body,
        grid=grid,
        in_specs=[
            pl.BlockSpec((_BLK, d), lambda i: (i, 0)),
            pl.BlockSpec(weights.shape, lambda i: (0, 0)),
        ],
        out_specs=pl.BlockSpec((_BLK,), lambda i: (i,)),
        out_shape=jax.ShapeDtypeStruct((b,), jnp.int32),
        compiler_params=pltpu.CompilerParams(
            dimension_semantics=("arbitrary",),
        ),
    )(x, weights)


# speedup vs baseline: 3.1792x; 1.2115x over previous
import functools

import jax
import jax.numpy as jnp
from jax import lax
from jax.experimental import pallas as pl
from jax.experimental.pallas import tpu as pltpu

_BLK = 8192


def _assign_body(x_ref, w_ref, out_ref):
    xb = x_ref[...]
    wb = w_ref[...]
    dots = lax.dot_general(wb, xb, (((1,), (1,)), ((), ())),
                           preferred_element_type=jnp.float32)
    w2 = jnp.sum(wb * wb, axis=1, keepdims=True)
    s = w2 - 2.0 * dots
    k = s.shape[0]
    min_s = jnp.min(s, axis=0, keepdims=True)
    iota_k = lax.broadcasted_iota(jnp.int32, s.shape, 0)
    idx = jnp.min(jnp.where(s == min_s, iota_k, k), axis=0)
    out_ref[...] = idx.astype(jnp.int32)


@jax.jit
def kernel(x, weights):
    b, d = x.shape
    grid = (b // _BLK,)
    return pl.pallas_call(
        _assign_body,
        grid=grid,
        in_specs=[
            pl.BlockSpec((_BLK, d), lambda i: (i, 0)),
            pl.BlockSpec(weights.shape, lambda i: (0, 0)),
        ],
        out_specs=pl.BlockSpec((_BLK,), lambda i: (i,)),
        out_shape=jax.ShapeDtypeStruct((b,), jnp.int32),
        compiler_params=pltpu.CompilerParams(
            dimension_semantics=("arbitrary",),
        ),
    )(x, weights)
